# baseline (speedup 1.0000x reference)
import functools

import jax
import jax.numpy as jnp
from jax import lax
from jax.experimental import pallas as pl
from jax.experimental.pallas import tpu as pltpu
from jax.experimental.pallas import tpu_sc as plsc

N = 10000
D = 256
E = 160000
NEG = 50000
NP = 10240
RB = 512
NBLK = NP // RB
EALL = E + N
NTILE = 32
NOWN = NP // NTILE
ACC2 = NOWN + 16
EPAD = 172032
WIN = 512
NWIN = EPAD // WIN
TREG = EPAD + WIN
NPAIR = E + NEG
PT = 6656
PPAD = NTILE * PT
AC = 64
LC = 64

f32 = jnp.float32
i32 = jnp.int32

_MESH = plsc.VectorSubcoreMesh(core_axis_name="c", subcore_axis_name="s")
_SC_PARAMS = pltpu.CompilerParams(needs_layout_passes=False)


def _pop(ok):
    return plsc.all_reduce_population_count(ok)[0]


@functools.partial(
    pl.kernel,
    out_type=[
        jax.ShapeDtypeStruct((NTILE * TREG,), i32),
        jax.ShapeDtypeStruct((NTILE, 16), i32),
        jax.ShapeDtypeStruct((NP,), i32),
    ],
    mesh=_MESH,
    compiler_params=_SC_PARAMS,
    scratch_types=[
        pltpu.VMEM((2, WIN), i32),
        pltpu.VMEM((2, WIN), i32),
        pltpu.VMEM((2 * WIN + 16,), i32),
        pltpu.VMEM((ACC2,), i32),
        pltpu.VMEM((16,), i32),
        pltpu.SemaphoreType.DMA,
        pltpu.SemaphoreType.DMA,
    ],
)
def _route_k(rows_hbm, cols_hbm, wl_hbm, tcnt_hbm, deg_hbm,
             rwin, cwin, stage, hist, pcnt, semr, semc):
    cc = lax.axis_index("c")
    ss = lax.axis_index("s")
    wid = ss * 2 + cc
    base = wid * NOWN
    iota16 = jnp.arange(16, dtype=i32)
    zero16 = jnp.zeros((16,), i32)

    def start_win(slot, w):
        woff = pl.multiple_of(w * WIN, 8)
        pltpu.async_copy(rows_hbm.at[pl.ds(woff, WIN)], rwin.at[slot], semr)
        pltpu.async_copy(cols_hbm.at[pl.ds(woff, WIN)], cwin.at[slot], semc)

    start_win(0, 0)

    def window(w, carry):
        ptr, flushes = carry
        slot = w % 2
        woff = pl.multiple_of(w * WIN, 8)
        pltpu.make_async_copy(rows_hbm.at[pl.ds(woff, WIN)],
                              rwin.at[slot], semr).wait()
        pltpu.make_async_copy(cols_hbm.at[pl.ds(woff, WIN)],
                              cwin.at[slot], semc).wait()

        @pl.when(w + 1 < NWIN)
        def _():
            start_win((w + 1) % 2, w + 1)

        def group(g, ptr2):
            r = rwin[slot, pl.ds(g * 16, 16)]
            cv = cwin[slot, pl.ds(g * 16, 16)]
            local = r - base
            ok = (local >= 0) & (local < NOWN)
            safe = jnp.where(ok, local, NOWN)
            packed = safe * 16384 + cv
            plsc.store_compressed(stage.at[pl.ds(ptr2, 16)], packed, mask=ok)
            return ptr2 + _pop(ok)

        ptr = lax.fori_loop(0, WIN // 16, group, ptr)

        @pl.when(ptr >= WIN)
        def _flush():
            dst = pl.multiple_of(wid * TREG + flushes * WIN, 8)
            pltpu.sync_copy(stage.at[pl.ds(0, WIN)], wl_hbm.at[pl.ds(dst, WIN)])
            for t in range(WIN // 16 + 1):
                tail = stage[pl.ds(WIN + t * 16, 16)]
                stage[pl.ds(t * 16, 16)] = tail

        did = jnp.where(ptr >= WIN, 1, 0).astype(i32)
        return (ptr - did * WIN, flushes + did)

    ptr, flushes = lax.fori_loop(
        0, NWIN, window, (jnp.zeros((), i32), jnp.zeros((), i32)))
    dst = pl.multiple_of(wid * TREG + flushes * WIN, 8)
    pltpu.sync_copy(stage.at[pl.ds(0, WIN)], wl_hbm.at[pl.ds(dst, WIN)])
    total = flushes * WIN + ptr
    pcnt[...] = total * jnp.where(iota16 == 0, 1, 0).astype(i32)
    pltpu.sync_copy(pcnt, tcnt_hbm.at[wid])

    def zb(i, carry):
        hist[pl.ds(i * 16, 16)] = zero16
        return carry

    lax.fori_loop(0, ACC2 // 16, zb, 0)
    nw2 = (total + WIN - 1) // WIN

    def hwin(w, carry):
        src = pl.multiple_of(wid * TREG + w * WIN, 8)
        pltpu.sync_copy(wl_hbm.at[pl.ds(src, WIN)], rwin.at[0])

        def hg(g, carry2):
            wv = rwin[0, pl.ds(g * 16, 16)]
            pos = w * WIN + g * 16 + iota16
            ok = pos < total
            loc = jnp.where(ok, wv >> 14, NOWN)
            cnts, lastm = plsc.scan_count(loc, mask=ok)
            plsc.addupdate_scatter(hist, [loc], cnts, mask=lastm)
            return carry2

        return lax.fori_loop(0, WIN // 16, hg, carry)

    lax.fori_loop(0, nw2, hwin, 0)
    pltpu.sync_copy(hist.at[pl.ds(0, NOWN)],
                    deg_hbm.at[pl.ds(pl.multiple_of(base, 8), NOWN)])


@functools.partial(
    pl.kernel,
    out_type=jax.ShapeDtypeStruct((NP, D), f32),
    mesh=_MESH,
    compiler_params=_SC_PARAMS,
    scratch_types=[
        pltpu.VMEM((WIN,), i32),
        pltpu.VMEM((2 * AC,), i32),
        pltpu.VMEM((2 * AC,), i32),
        pltpu.VMEM((16,), i32),
        pltpu.VMEM((2, AC, D), f32),
        pltpu.VMEM((ACC2, D), f32),
        pltpu.SemaphoreType.DMA,
    ],
)
def _agg_k(hhat_hbm, wl_hbm, tcnt_hbm, zsrc_hbm, out_hbm,
           wbuf, cbuf, lbuf, cntv, gbuf, acc, sem):
    cc = lax.axis_index("c")
    ss = lax.axis_index("s")
    wid = ss * 2 + cc
    iota16 = jnp.arange(16, dtype=i32)
    pltpu.sync_copy(zsrc_hbm, acc)
    pltpu.sync_copy(tcnt_hbm.at[wid], cntv)
    cnt = cntv[pl.ds(0, 16)][0]
    nch = (cnt + (AC - 1)) // AC
    perblk = WIN // AC

    def load_block(j):
        src = pl.multiple_of(wid * TREG + (j // perblk) * WIN, 8)
        pltpu.sync_copy(wl_hbm.at[pl.ds(src, WIN)], wbuf)

    def prep_start(slot, j):
        k0 = (j % perblk) * AC
        for k in range(AC // 16):
            w = wbuf[pl.ds(k0 + k * 16, 16)]
            pos = j * AC + k * 16 + iota16
            ok = pos < cnt
            cbuf[pl.ds(slot * AC + k * 16, 16)] = jnp.where(
                ok, w & 16383, k * 16 + iota16)
            lbuf[pl.ds(slot * AC + k * 16, 16)] = jnp.where(
                ok, w >> 14, NOWN + (iota16 & 7))
        pltpu.async_copy(hhat_hbm.at[cbuf.at[pl.ds(slot * AC, AC)]],
                         gbuf.at[slot], sem)

    @pl.when(nch > 0)
    def _():
        load_block(0)
        prep_start(0, 0)

    def body(j, carry):
        slot = j % 2
        pltpu.make_async_copy(hhat_hbm.at[cbuf.at[pl.ds(slot * AC, AC)]],
                              gbuf.at[slot], sem).wait()

        @pl.when(((j + 1) % perblk == 0) & (j + 1 < nch))
        def _():
            load_block(j + 1)

        @pl.when(j + 1 < nch)
        def _():
            prep_start((j + 1) % 2, j + 1)

        for k in range(AC // 16):
            lv = lbuf[pl.ds(slot * AC + k * 16, 16)]
            for p in range(16):
                row = lv[p]
                e = k * 16 + p
                for v in range(D // 16):
                    plsc.addupdate(acc.at[row, pl.ds(v * 16, 16)],
                                   gbuf[slot, e, pl.ds(v * 16, 16)])
        return carry

    lax.fori_loop(0, nch, body, 0)
    pltpu.sync_copy(acc.at[pl.ds(0, NOWN)],
                    out_hbm.at[pl.ds(pl.multiple_of(wid * NOWN, 8), NOWN)])


@functools.partial(
    pl.kernel,
    out_type=jax.ShapeDtypeStruct((NTILE, 32), f32),
    mesh=_MESH,
    compiler_params=_SC_PARAMS,
    scratch_types=[
        pltpu.VMEM((WIN,), i32),
        pltpu.VMEM((WIN,), i32),
        pltpu.VMEM((PT + 16,), i32),
        pltpu.VMEM((2 * LC,), i32),
        pltpu.VMEM((2 * LC,), i32),
        pltpu.VMEM((2, LC, D), f32),
        pltpu.VMEM((2, LC, D), f32),
        pltpu.VMEM((32,), f32),
        pltpu.SemaphoreType.DMA,
        pltpu.SemaphoreType.DMA,
    ],
)
def _loss_k(rep_hbm, pa_hbm, pb_hbm, out_hbm,
            paw, pbw, pwords, abuf, bbuf, bufa, bufb, pout, sema, semb):
    cc = lax.axis_index("c")
    ss = lax.axis_index("s")
    wid = ss * 2 + cc
    iota16 = jnp.arange(16, dtype=i32)
    ones16 = jnp.ones((16,), f32)
    zeros16 = jnp.zeros((16,), f32)

    def win(w, ptr):
        woff = pl.multiple_of(wid * PT + w * WIN, 8)
        pltpu.sync_copy(pa_hbm.at[pl.ds(woff, WIN)], paw)
        pltpu.sync_copy(pb_hbm.at[pl.ds(woff, WIN)], pbw)

        def grp(g, ptr2):
            av = paw[pl.ds(g * 16, 16)]
            bv = pbw[pl.ds(g * 16, 16)]
            gpos = wid * PT + w * WIN + g * 16 + iota16
            ok = av < bv
            word = av * 32768 + bv * 2 + jnp.where(gpos < E, 1, 0).astype(i32)
            plsc.store_compressed(pwords.at[pl.ds(ptr2, 16)], word, mask=ok)
            return ptr2 + _pop(ok)

        return lax.fori_loop(0, WIN // 16, grp, ptr)

    cnt = lax.fori_loop(0, PT // WIN, win, jnp.zeros((), i32))
    nch = (cnt + (LC - 1)) // LC

    def prep_start(slot, j):
        for k in range(LC // 16):
            w = pwords[pl.ds(j * LC + k * 16, 16)]
            pos = j * LC + k * 16 + iota16
            ok = pos < cnt
            abuf[pl.ds(slot * LC + k * 16, 16)] = jnp.where(
                ok, w >> 15, k * 16 + iota16)
            bbuf[pl.ds(slot * LC + k * 16, 16)] = jnp.where(
                ok, (w >> 1) & 16383, k * 16 + iota16)
        pltpu.async_copy(rep_hbm.at[abuf.at[pl.ds(slot * LC, LC)]],
                         bufa.at[slot], sema)
        pltpu.async_copy(rep_hbm.at[bbuf.at[pl.ds(slot * LC, LC)]],
                         bufb.at[slot], semb)

    @pl.when(nch > 0)
    def _():
        prep_start(0, 0)

    def body(j, sqc):
        slot = j % 2
        pltpu.make_async_copy(rep_hbm.at[abuf.at[pl.ds(slot * LC, LC)]],
                              bufa.at[slot], sema).wait()
        pltpu.make_async_copy(rep_hbm.at[bbuf.at[pl.ds(slot * LC, LC)]],
                              bufb.at[slot], semb).wait()

        @pl.when(j + 1 < nch)
        def _():
            prep_start((j + 1) % 2, j + 1)

        def group(g, sqc2):
            dotv = zeros16
            for p in range(16):
                row = g * 16 + p
                acc = (bufa[slot, row, pl.ds(0, 16)]
                       * bufb[slot, row, pl.ds(0, 16)])
                for v in range(1, D // 16):
                    acc = acc + (bufa[slot, row, pl.ds(v * 16, 16)]
                                 * bufb[slot, row, pl.ds(v * 16, 16)])
                dot = jnp.sum(acc)
                onehot = jnp.where(iota16 == p, f32(1.0), f32(0.0))
                dotv = dotv + dot * onehot
            w = pwords[pl.ds(j * LC + g * 16, 16)]
            pos = j * LC + g * 16 + iota16
            okv = jnp.where(pos < cnt, ones16, zeros16)
            tv = (w & 1).astype(f32)
            dv = dotv - tv
            return sqc2 + okv * dv * dv

        return lax.fori_loop(0, LC // 16, group, sqc)

    sqv = lax.fori_loop(0, nch, body, zeros16)
    pout[pl.ds(0, 16)] = sqv
    pout[pl.ds(16, 16)] = cnt.astype(f32) * jnp.where(iota16 == 0,
                                                      f32(1.0), f32(0.0))
    pltpu.sync_copy(pout, out_hbm.at[wid])


def _enc_body(deg_ref, x_ref, w_ref, b_ref, hh_ref, dinv_ref):
    deg = deg_ref[...].astype(f32)
    dinv = jnp.where(deg > 0, lax.rsqrt(deg), f32(0.0))
    h = jnp.dot(x_ref[...], w_ref[...], preferred_element_type=f32) + b_ref[...]
    hh_ref[...] = h * dinv[:, None]
    dinv_ref[...] = dinv[:, None]


def _mid_body(s1_ref, dinv_ref, w_ref, b_ref, hh_ref):
    dinv = dinv_ref[...]
    h = jnp.maximum(s1_ref[...] * dinv, f32(0.0))
    h2 = jnp.dot(h, w_ref[...], preferred_element_type=f32) + b_ref[...]
    hh_ref[...] = h2 * dinv


def _norm_body(s2_ref, dinv_ref, rep_ref):
    o = s2_ref[...] * dinv_ref[...]
    nrm = jnp.sqrt(jnp.sum(o * o, axis=1, keepdims=True))
    rep_ref[...] = o / jnp.maximum(nrm, f32(1e-12))


def _fin_body(p_ref, o_ref):
    p = p_ref[...]
    lane = lax.broadcasted_iota(i32, p.shape, 1)
    sq = jnp.sum(jnp.where(lane < 16, p, f32(0.0)))
    m = jnp.sum(jnp.where(lane >= 16, p, f32(0.0)))
    o_ref[...] = jnp.reshape(sq * f32(N) / m, (1, 1))


_enc = pl.pallas_call(
    _enc_body,
    grid=(NBLK,),
    in_specs=[
        pl.BlockSpec((RB,), lambda i: (i,)),
        pl.BlockSpec((RB, D), lambda i: (i, 0)),
        pl.BlockSpec((D, D), lambda i: (0, 0)),
        pl.BlockSpec((1, D), lambda i: (0, 0)),
    ],
    out_specs=[
        pl.BlockSpec((RB, D), lambda i: (i, 0)),
        pl.BlockSpec((RB, 1), lambda i: (i, 0)),
    ],
    out_shape=[
        jax.ShapeDtypeStruct((NP, D), f32),
        jax.ShapeDtypeStruct((NP, 1), f32),
    ],
)

_mid = pl.pallas_call(
    _mid_body,
    grid=(NBLK,),
    in_specs=[
        pl.BlockSpec((RB, D), lambda i: (i, 0)),
        pl.BlockSpec((RB, 1), lambda i: (i, 0)),
        pl.BlockSpec((D, D), lambda i: (0, 0)),
        pl.BlockSpec((1, D), lambda i: (0, 0)),
    ],
    out_specs=pl.BlockSpec((RB, D), lambda i: (i, 0)),
    out_shape=jax.ShapeDtypeStruct((NP, D), f32),
)

_normk = pl.pallas_call(
    _norm_body,
    grid=(NBLK,),
    in_specs=[
        pl.BlockSpec((RB, D), lambda i: (i, 0)),
        pl.BlockSpec((RB, 1), lambda i: (i, 0)),
    ],
    out_specs=pl.BlockSpec((RB, D), lambda i: (i, 0)),
    out_shape=jax.ShapeDtypeStruct((NP, D), f32),
)

_fin = pl.pallas_call(
    _fin_body,
    in_specs=[pl.BlockSpec((NTILE, 32), lambda: (0, 0))],
    out_specs=pl.BlockSpec((1, 1), lambda: (0, 0)),
    out_shape=jax.ShapeDtypeStruct((1, 1), f32),
)


def kernel(features, edge_index, W1, b1, W2, b2):
    x = jnp.pad(features, ((0, NP - N), (0, 0)))
    loop = jnp.arange(N, dtype=i32)
    epad = EPAD - EALL
    rows = jnp.concatenate(
        [edge_index[0], loop, jnp.full((epad,), -1, i32)])
    cols = jnp.concatenate(
        [edge_index[1], loop, jnp.zeros((epad,), i32)])
    zsrc = jnp.zeros((ACC2, D), f32)

    wl, tcnt, deg = _route_k(rows, cols)
    hhat1, dinv = _enc(deg, x, W1, b1.reshape(1, D))
    s1 = _agg_k(hhat1, wl, tcnt, zsrc)
    hhat2 = _mid(s1, dinv, W2, b2.reshape(1, D))
    s2 = _agg_k(hhat2, wl, tcnt, zsrc)
    rep_full = _normk(s2, dinv)

    randn = jax.random.randint(jax.random.key(42), (2, NEG), 0, N, dtype=i32)
    ppad = PPAD - NPAIR
    spread = (jnp.arange(ppad, dtype=i32) * 53) % N
    pa = jnp.concatenate([edge_index[0], randn[0], spread])
    pb = jnp.concatenate([edge_index[1], randn[1], spread])

    partials = _loss_k(rep_full, pa, pb)
    loss = _fin(partials)[0, 0]
    return rep_full[:N], loss


# speedup vs baseline: 1.1973x; 1.1973x over previous
import functools

import jax
import jax.numpy as jnp
from jax import lax
from jax.experimental import pallas as pl
from jax.experimental.pallas import tpu as pltpu
from jax.experimental.pallas import tpu_sc as plsc

N = 10000
D = 256
E = 160000
NEG = 50000
NP = 10240
RB = 512
NBLK = NP // RB
EALL = E + N
NTILE = 32
NOWN = NP // NTILE
ACC2 = NOWN + 16
EPAD = 172032
WIN = 512
NWIN = EPAD // WIN
TREG = EPAD + WIN
NPAIR = E + NEG
PT = 6656
PPAD = NTILE * PT
AC = 80
LC = 96

f32 = jnp.float32
i32 = jnp.int32

_MESH = plsc.VectorSubcoreMesh(core_axis_name="c", subcore_axis_name="s")
_SC_PARAMS = pltpu.CompilerParams(needs_layout_passes=False)


def _pop(ok):
    return plsc.all_reduce_population_count(ok)[0]


@functools.partial(
    pl.kernel,
    out_type=[
        jax.ShapeDtypeStruct((NTILE * TREG,), i32),
        jax.ShapeDtypeStruct((NTILE, 16), i32),
        jax.ShapeDtypeStruct((NP,), i32),
    ],
    mesh=_MESH,
    compiler_params=_SC_PARAMS,
    scratch_types=[
        pltpu.VMEM((2, WIN), i32),
        pltpu.VMEM((2, WIN), i32),
        pltpu.VMEM((2 * WIN + 16,), i32),
        pltpu.VMEM((ACC2,), i32),
        pltpu.VMEM((16,), i32),
        pltpu.SemaphoreType.DMA,
        pltpu.SemaphoreType.DMA,
    ],
)
def _route_k(rows_hbm, cols_hbm, wl_hbm, tcnt_hbm, deg_hbm,
             rwin, cwin, stage, hist, pcnt, semr, semc):
    cc = lax.axis_index("c")
    ss = lax.axis_index("s")
    wid = ss * 2 + cc
    base = wid * NOWN
    iota16 = jnp.arange(16, dtype=i32)
    zero16 = jnp.zeros((16,), i32)

    def start_win(slot, w):
        woff = pl.multiple_of(w * WIN, 8)
        pltpu.async_copy(rows_hbm.at[pl.ds(woff, WIN)], rwin.at[slot], semr)
        pltpu.async_copy(cols_hbm.at[pl.ds(woff, WIN)], cwin.at[slot], semc)

    start_win(0, 0)

    def window(w, carry):
        ptr, flushes = carry
        slot = w % 2
        woff = pl.multiple_of(w * WIN, 8)
        pltpu.make_async_copy(rows_hbm.at[pl.ds(woff, WIN)],
                              rwin.at[slot], semr).wait()
        pltpu.make_async_copy(cols_hbm.at[pl.ds(woff, WIN)],
                              cwin.at[slot], semc).wait()

        @pl.when(w + 1 < NWIN)
        def _():
            start_win((w + 1) % 2, w + 1)

        def group(g, ptr2):
            r = rwin[slot, pl.ds(g * 16, 16)]
            cv = cwin[slot, pl.ds(g * 16, 16)]
            local = r - base
            ok = (local >= 0) & (local < NOWN)
            safe = jnp.where(ok, local, NOWN)
            packed = safe * 16384 + cv
            plsc.store_compressed(stage.at[pl.ds(ptr2, 16)], packed, mask=ok)
            return ptr2 + _pop(ok)

        ptr = lax.fori_loop(0, WIN // 16, group, ptr)

        @pl.when(ptr >= WIN)
        def _flush():
            dst = pl.multiple_of(wid * TREG + flushes * WIN, 8)
            pltpu.sync_copy(stage.at[pl.ds(0, WIN)], wl_hbm.at[pl.ds(dst, WIN)])
            for t in range(WIN // 16 + 1):
                tail = stage[pl.ds(WIN + t * 16, 16)]
                stage[pl.ds(t * 16, 16)] = tail

        did = jnp.where(ptr >= WIN, 1, 0).astype(i32)
        return (ptr - did * WIN, flushes + did)

    ptr, flushes = lax.fori_loop(
        0, NWIN, window, (jnp.zeros((), i32), jnp.zeros((), i32)))
    dst = pl.multiple_of(wid * TREG + flushes * WIN, 8)
    pltpu.sync_copy(stage.at[pl.ds(0, WIN)], wl_hbm.at[pl.ds(dst, WIN)])
    total = flushes * WIN + ptr
    pcnt[...] = total * jnp.where(iota16 == 0, 1, 0).astype(i32)
    pltpu.sync_copy(pcnt, tcnt_hbm.at[wid])

    def zb(i, carry):
        hist[pl.ds(i * 16, 16)] = zero16
        return carry

    lax.fori_loop(0, ACC2 // 16, zb, 0)
    nw2 = (total + WIN - 1) // WIN

    def hwin(w, carry):
        src = pl.multiple_of(wid * TREG + w * WIN, 8)
        pltpu.sync_copy(wl_hbm.at[pl.ds(src, WIN)], rwin.at[0])

        def hg(g, carry2):
            wv = rwin[0, pl.ds(g * 16, 16)]
            pos = w * WIN + g * 16 + iota16
            ok = pos < total
            loc = jnp.where(ok, wv >> 14, NOWN)
            cnts, lastm = plsc.scan_count(loc, mask=ok)
            plsc.addupdate_scatter(hist, [loc], cnts, mask=lastm)
            return carry2

        return lax.fori_loop(0, WIN // 16, hg, carry)

    lax.fori_loop(0, nw2, hwin, 0)
    pltpu.sync_copy(hist.at[pl.ds(0, NOWN)],
                    deg_hbm.at[pl.ds(pl.multiple_of(base, 8), NOWN)])


@functools.partial(
    pl.kernel,
    out_type=jax.ShapeDtypeStruct((NP, D), f32),
    mesh=_MESH,
    compiler_params=_SC_PARAMS,
    scratch_types=[
        pltpu.VMEM((8 * AC,), i32),
        pltpu.VMEM((2 * AC,), i32),
        pltpu.VMEM((2 * AC,), i32),
        pltpu.VMEM((16,), i32),
        pltpu.VMEM((2, AC, D), f32),
        pltpu.VMEM((ACC2, D), f32),
        pltpu.SemaphoreType.DMA,
    ],
)
def _agg_k(hhat_hbm, wl_hbm, tcnt_hbm, zsrc_hbm, out_hbm,
           wbuf, cbuf, lbuf, cntv, gbuf, acc, sem):
    cc = lax.axis_index("c")
    ss = lax.axis_index("s")
    wid = ss * 2 + cc
    iota16 = jnp.arange(16, dtype=i32)
    pltpu.sync_copy(zsrc_hbm, acc)
    pltpu.sync_copy(tcnt_hbm.at[wid], cntv)
    cnt = cntv[pl.ds(0, 16)][0]
    nch = (cnt + (AC - 1)) // AC
    perblk = 8

    def load_block(j):
        src = pl.multiple_of(wid * TREG + (j // perblk) * (perblk * AC), 8)
        pltpu.sync_copy(wl_hbm.at[pl.ds(src, perblk * AC)], wbuf)

    def prep_start(slot, j):
        k0 = (j % perblk) * AC
        for k in range(AC // 16):
            w = wbuf[pl.ds(k0 + k * 16, 16)]
            pos = j * AC + k * 16 + iota16
            ok = pos < cnt
            cbuf[pl.ds(slot * AC + k * 16, 16)] = jnp.where(
                ok, w & 16383, k * 16 + iota16)
            lbuf[pl.ds(slot * AC + k * 16, 16)] = jnp.where(
                ok, w >> 14, NOWN + (iota16 & 7))
        pltpu.async_copy(hhat_hbm.at[cbuf.at[pl.ds(slot * AC, AC)]],
                         gbuf.at[slot], sem)

    @pl.when(nch > 0)
    def _():
        load_block(0)
        prep_start(0, 0)

    def body(j, carry):
        slot = j % 2
        pltpu.make_async_copy(hhat_hbm.at[cbuf.at[pl.ds(slot * AC, AC)]],
                              gbuf.at[slot], sem).wait()

        @pl.when(((j + 1) % perblk == 0) & (j + 1 < nch))
        def _():
            load_block(j + 1)

        @pl.when(j + 1 < nch)
        def _():
            prep_start((j + 1) % 2, j + 1)

        def grp(k, carry2):
            lv = lbuf[pl.ds(slot * AC + k * 16, 16)]
            for p in range(16):
                row = lv[p]
                e = k * 16 + p
                for v in range(D // 16):
                    plsc.addupdate(acc.at[row, pl.ds(v * 16, 16)],
                                   gbuf[slot, e, pl.ds(v * 16, 16)])
            return carry2

        lax.fori_loop(0, AC // 16, grp, 0)
        return carry

    lax.fori_loop(0, nch, body, 0)
    pltpu.sync_copy(acc.at[pl.ds(0, NOWN)],
                    out_hbm.at[pl.ds(pl.multiple_of(wid * NOWN, 8), NOWN)])


@functools.partial(
    pl.kernel,
    out_type=jax.ShapeDtypeStruct((NTILE, 32), f32),
    mesh=_MESH,
    compiler_params=_SC_PARAMS,
    scratch_types=[
        pltpu.VMEM((WIN,), i32),
        pltpu.VMEM((WIN,), i32),
        pltpu.VMEM((PT + 16,), i32),
        pltpu.VMEM((2 * LC,), i32),
        pltpu.VMEM((2 * LC,), i32),
        pltpu.VMEM((2, LC, D), f32),
        pltpu.VMEM((2, LC, D), f32),
        pltpu.VMEM((32,), f32),
        pltpu.SemaphoreType.DMA,
        pltpu.SemaphoreType.DMA,
    ],
)
def _loss_k(rep_hbm, pa_hbm, pb_hbm, out_hbm,
            paw, pbw, pwords, abuf, bbuf, bufa, bufb, pout, sema, semb):
    cc = lax.axis_index("c")
    ss = lax.axis_index("s")
    wid = ss * 2 + cc
    iota16 = jnp.arange(16, dtype=i32)
    ones16 = jnp.ones((16,), f32)
    zeros16 = jnp.zeros((16,), f32)

    def win(w, ptr):
        woff = pl.multiple_of(wid * PT + w * WIN, 8)
        pltpu.sync_copy(pa_hbm.at[pl.ds(woff, WIN)], paw)
        pltpu.sync_copy(pb_hbm.at[pl.ds(woff, WIN)], pbw)

        def grp(g, ptr2):
            av = paw[pl.ds(g * 16, 16)]
            bv = pbw[pl.ds(g * 16, 16)]
            gpos = wid * PT + w * WIN + g * 16 + iota16
            ok = av < bv
            word = av * 32768 + bv * 2 + jnp.where(gpos < E, 1, 0).astype(i32)
            plsc.store_compressed(pwords.at[pl.ds(ptr2, 16)], word, mask=ok)
            return ptr2 + _pop(ok)

        return lax.fori_loop(0, WIN // 16, grp, ptr)

    cnt = lax.fori_loop(0, PT // WIN, win, jnp.zeros((), i32))
    nch = (cnt + (LC - 1)) // LC

    def prep_start(slot, j):
        for k in range(LC // 16):
            w = pwords[pl.ds(j * LC + k * 16, 16)]
            pos = j * LC + k * 16 + iota16
            ok = pos < cnt
            abuf[pl.ds(slot * LC + k * 16, 16)] = jnp.where(
                ok, w >> 15, k * 16 + iota16)
            bbuf[pl.ds(slot * LC + k * 16, 16)] = jnp.where(
                ok, (w >> 1) & 16383, k * 16 + iota16)
        pltpu.async_copy(rep_hbm.at[abuf.at[pl.ds(slot * LC, LC)]],
                         bufa.at[slot], sema)
        pltpu.async_copy(rep_hbm.at[bbuf.at[pl.ds(slot * LC, LC)]],
                         bufb.at[slot], semb)

    @pl.when(nch > 0)
    def _():
        prep_start(0, 0)

    def body(j, sqc):
        slot = j % 2
        pltpu.make_async_copy(rep_hbm.at[abuf.at[pl.ds(slot * LC, LC)]],
                              bufa.at[slot], sema).wait()
        pltpu.make_async_copy(rep_hbm.at[bbuf.at[pl.ds(slot * LC, LC)]],
                              bufb.at[slot], semb).wait()

        @pl.when(j + 1 < nch)
        def _():
            prep_start((j + 1) % 2, j + 1)

        def group(g, sqc2):
            dotv = zeros16
            for p in range(16):
                row = g * 16 + p
                acc = (bufa[slot, row, pl.ds(0, 16)]
                       * bufb[slot, row, pl.ds(0, 16)])
                for v in range(1, D // 16):
                    acc = acc + (bufa[slot, row, pl.ds(v * 16, 16)]
                                 * bufb[slot, row, pl.ds(v * 16, 16)])
                dot = jnp.sum(acc)
                onehot = jnp.where(iota16 == p, f32(1.0), f32(0.0))
                dotv = dotv + dot * onehot
            w = pwords[pl.ds(j * LC + g * 16, 16)]
            pos = j * LC + g * 16 + iota16
            okv = jnp.where(pos < cnt, ones16, zeros16)
            tv = (w & 1).astype(f32)
            dv = dotv - tv
            return sqc2 + okv * dv * dv

        return lax.fori_loop(0, LC // 16, group, sqc)

    sqv = lax.fori_loop(0, nch, body, zeros16)
    pout[pl.ds(0, 16)] = sqv
    pout[pl.ds(16, 16)] = cnt.astype(f32) * jnp.where(iota16 == 0,
                                                      f32(1.0), f32(0.0))
    pltpu.sync_copy(pout, out_hbm.at[wid])


def _enc_body(deg_ref, x_ref, w_ref, b_ref, hh_ref, dinv_ref):
    deg = deg_ref[...].astype(f32)
    dinv = jnp.where(deg > 0, lax.rsqrt(deg), f32(0.0))
    h = jnp.dot(x_ref[...], w_ref[...], preferred_element_type=f32) + b_ref[...]
    hh_ref[...] = h * dinv[:, None]
    dinv_ref[...] = dinv[:, None]


def _mid_body(s1_ref, dinv_ref, w_ref, b_ref, hh_ref):
    dinv = dinv_ref[...]
    h = jnp.maximum(s1_ref[...] * dinv, f32(0.0))
    h2 = jnp.dot(h, w_ref[...], preferred_element_type=f32) + b_ref[...]
    hh_ref[...] = h2 * dinv


def _norm_body(s2_ref, dinv_ref, rep_ref):
    o = s2_ref[...] * dinv_ref[...]
    nrm = jnp.sqrt(jnp.sum(o * o, axis=1, keepdims=True))
    rep_ref[...] = o / jnp.maximum(nrm, f32(1e-12))


def _fin_body(p_ref, o_ref):
    p = p_ref[...]
    lane = lax.broadcasted_iota(i32, p.shape, 1)
    sq = jnp.sum(jnp.where(lane < 16, p, f32(0.0)))
    m = jnp.sum(jnp.where(lane >= 16, p, f32(0.0)))
    o_ref[...] = jnp.reshape(sq * f32(N) / m, (1, 1))


_enc = pl.pallas_call(
    _enc_body,
    grid=(NBLK,),
    in_specs=[
        pl.BlockSpec((RB,), lambda i: (i,)),
        pl.BlockSpec((RB, D), lambda i: (i, 0)),
        pl.BlockSpec((D, D), lambda i: (0, 0)),
        pl.BlockSpec((1, D), lambda i: (0, 0)),
    ],
    out_specs=[
        pl.BlockSpec((RB, D), lambda i: (i, 0)),
        pl.BlockSpec((RB, 1), lambda i: (i, 0)),
    ],
    out_shape=[
        jax.ShapeDtypeStruct((NP, D), f32),
        jax.ShapeDtypeStruct((NP, 1), f32),
    ],
)

_mid = pl.pallas_call(
    _mid_body,
    grid=(NBLK,),
    in_specs=[
        pl.BlockSpec((RB, D), lambda i: (i, 0)),
        pl.BlockSpec((RB, 1), lambda i: (i, 0)),
        pl.BlockSpec((D, D), lambda i: (0, 0)),
        pl.BlockSpec((1, D), lambda i: (0, 0)),
    ],
    out_specs=pl.BlockSpec((RB, D), lambda i: (i, 0)),
    out_shape=jax.ShapeDtypeStruct((NP, D), f32),
)

_normk = pl.pallas_call(
    _norm_body,
    grid=(NBLK,),
    in_specs=[
        pl.BlockSpec((RB, D), lambda i: (i, 0)),
        pl.BlockSpec((RB, 1), lambda i: (i, 0)),
    ],
    out_specs=pl.BlockSpec((RB, D), lambda i: (i, 0)),
    out_shape=jax.ShapeDtypeStruct((NP, D), f32),
)

_fin = pl.pallas_call(
    _fin_body,
    in_specs=[pl.BlockSpec((NTILE, 32), lambda: (0, 0))],
    out_specs=pl.BlockSpec((1, 1), lambda: (0, 0)),
    out_shape=jax.ShapeDtypeStruct((1, 1), f32),
)


def kernel(features, edge_index, W1, b1, W2, b2):
    x = jnp.pad(features, ((0, NP - N), (0, 0)))
    loop = jnp.arange(N, dtype=i32)
    epad = EPAD - EALL
    rows = jnp.concatenate(
        [edge_index[0], loop, jnp.full((epad,), -1, i32)])
    cols = jnp.concatenate(
        [edge_index[1], loop, jnp.zeros((epad,), i32)])
    zsrc = jnp.zeros((ACC2, D), f32)

    wl, tcnt, deg = _route_k(rows, cols)
    hhat1, dinv = _enc(deg, x, W1, b1.reshape(1, D))
    s1 = _agg_k(hhat1, wl, tcnt, zsrc)
    hhat2 = _mid(s1, dinv, W2, b2.reshape(1, D))
    s2 = _agg_k(hhat2, wl, tcnt, zsrc)
    rep_full = _normk(s2, dinv)

    randn = jax.random.randint(jax.random.key(42), (2, NEG), 0, N, dtype=i32)
    ppad = PPAD - NPAIR
    spread = (jnp.arange(ppad, dtype=i32) * 53) % N
    pa = jnp.concatenate([edge_index[0], randn[0], spread])
    pb = jnp.concatenate([edge_index[1], randn[1], spread])

    partials = _loss_k(rep_full, pa, pb)
    loss = _fin(partials)[0, 0]
    return rep_full[:N], loss


# speedup vs baseline: 1.2890x; 1.0766x over previous
import functools

import jax
import jax.numpy as jnp
from jax import lax
from jax.experimental import pallas as pl
from jax.experimental.pallas import tpu as pltpu
from jax.experimental.pallas import tpu_sc as plsc

N = 10000
D = 256
E = 160000
NEG = 50000
NP = 10240
RB = 512
NBLK = NP // RB
EALL = E + N
NTILE = 32
NOWN = NP // NTILE
ACC2 = NOWN + 16
EPAD = 172032
WIN = 512
RWIN = 1024
NWIN = EPAD // RWIN
TREG = EPAD + RWIN
NPAIR = E + NEG
PT = 6656
PPAD = NTILE * PT
AC = 80
LC = 96

f32 = jnp.float32
i32 = jnp.int32

_MESH = plsc.VectorSubcoreMesh(core_axis_name="c", subcore_axis_name="s")
_SC_PARAMS = pltpu.CompilerParams(needs_layout_passes=False)


def _pop(ok):
    return plsc.all_reduce_population_count(ok)[0]


@functools.partial(
    pl.kernel,
    out_type=[
        jax.ShapeDtypeStruct((NTILE * TREG,), i32),
        jax.ShapeDtypeStruct((NTILE, 16), i32),
        jax.ShapeDtypeStruct((NP,), i32),
    ],
    mesh=_MESH,
    compiler_params=_SC_PARAMS,
    scratch_types=[
        pltpu.VMEM((2, RWIN), i32),
        pltpu.VMEM((2, RWIN), i32),
        pltpu.VMEM((2 * RWIN + 16,), i32),
        pltpu.VMEM((ACC2,), i32),
        pltpu.VMEM((16,), i32),
        pltpu.SemaphoreType.DMA,
        pltpu.SemaphoreType.DMA,
    ],
)
def _route_k(rows_hbm, cols_hbm, wl_hbm, tcnt_hbm, deg_hbm,
             rwin, cwin, stage, hist, pcnt, semr, semc):
    cc = lax.axis_index("c")
    ss = lax.axis_index("s")
    wid = ss * 2 + cc
    base = wid * NOWN
    iota16 = jnp.arange(16, dtype=i32)
    zero16 = jnp.zeros((16,), i32)

    def start_win(slot, w):
        woff = pl.multiple_of(w * RWIN, 8)
        pltpu.async_copy(rows_hbm.at[pl.ds(woff, RWIN)], rwin.at[slot], semr)
        pltpu.async_copy(cols_hbm.at[pl.ds(woff, RWIN)], cwin.at[slot], semc)

    start_win(0, 0)

    def window(w, carry):
        ptr, flushes = carry
        slot = w % 2
        woff = pl.multiple_of(w * RWIN, 8)
        pltpu.make_async_copy(rows_hbm.at[pl.ds(woff, RWIN)],
                              rwin.at[slot], semr).wait()
        pltpu.make_async_copy(cols_hbm.at[pl.ds(woff, RWIN)],
                              cwin.at[slot], semc).wait()

        @pl.when(w + 1 < NWIN)
        def _():
            start_win((w + 1) % 2, w + 1)

        def group(g, ptr2):
            r = rwin[slot, pl.ds(g * 16, 16)]
            cv = cwin[slot, pl.ds(g * 16, 16)]
            local = r - base
            ok = (local >= 0) & (local < NOWN)
            safe = jnp.where(ok, local, NOWN)
            packed = safe * 16384 + cv
            plsc.store_compressed(stage.at[pl.ds(ptr2, 16)], packed, mask=ok)
            return ptr2 + _pop(ok)

        ptr = lax.fori_loop(0, RWIN // 16, group, ptr)

        @pl.when(ptr >= RWIN)
        def _flush():
            dst = pl.multiple_of(wid * TREG + flushes * RWIN, 8)
            pltpu.sync_copy(stage.at[pl.ds(0, RWIN)], wl_hbm.at[pl.ds(dst, RWIN)])
            for t in range(RWIN // 16 + 1):
                tail = stage[pl.ds(RWIN + t * 16, 16)]
                stage[pl.ds(t * 16, 16)] = tail

        did = jnp.where(ptr >= RWIN, 1, 0).astype(i32)
        return (ptr - did * RWIN, flushes + did)

    ptr, flushes = lax.fori_loop(
        0, NWIN, window, (jnp.zeros((), i32), jnp.zeros((), i32)))
    dst = pl.multiple_of(wid * TREG + flushes * RWIN, 8)
    pltpu.sync_copy(stage.at[pl.ds(0, RWIN)], wl_hbm.at[pl.ds(dst, RWIN)])
    total = flushes * RWIN + ptr
    pcnt[...] = total * jnp.where(iota16 == 0, 1, 0).astype(i32)
    pltpu.sync_copy(pcnt, tcnt_hbm.at[wid])

    def zb(i, carry):
        hist[pl.ds(i * 16, 16)] = zero16
        return carry

    lax.fori_loop(0, ACC2 // 16, zb, 0)
    nw2 = (total + RWIN - 1) // RWIN

    def hwin(w, carry):
        src = pl.multiple_of(wid * TREG + w * RWIN, 8)
        pltpu.sync_copy(wl_hbm.at[pl.ds(src, RWIN)], rwin.at[0])

        def hg(g, carry2):
            wv = rwin[0, pl.ds(g * 16, 16)]
            pos = w * RWIN + g * 16 + iota16
            ok = pos < total
            loc = jnp.where(ok, wv >> 14, NOWN)
            cnts, lastm = plsc.scan_count(loc, mask=ok)
            plsc.addupdate_scatter(hist, [loc], cnts, mask=lastm)
            return carry2

        return lax.fori_loop(0, RWIN // 16, hg, carry)

    lax.fori_loop(0, nw2, hwin, 0)
    pltpu.sync_copy(hist.at[pl.ds(0, NOWN)],
                    deg_hbm.at[pl.ds(pl.multiple_of(base, 8), NOWN)])


@functools.partial(
    pl.kernel,
    out_type=jax.ShapeDtypeStruct((NP, D), f32),
    mesh=_MESH,
    compiler_params=_SC_PARAMS,
    scratch_types=[
        pltpu.VMEM((8 * AC,), i32),
        pltpu.VMEM((2 * AC,), i32),
        pltpu.VMEM((2 * AC,), i32),
        pltpu.VMEM((16,), i32),
        pltpu.VMEM((2, AC, D), f32),
        pltpu.VMEM((ACC2, D), f32),
        pltpu.SemaphoreType.DMA,
    ],
)
def _agg_k(hhat_hbm, wl_hbm, tcnt_hbm, zsrc_hbm, out_hbm,
           wbuf, cbuf, lbuf, cntv, gbuf, acc, sem):
    cc = lax.axis_index("c")
    ss = lax.axis_index("s")
    wid = ss * 2 + cc
    iota16 = jnp.arange(16, dtype=i32)
    pltpu.sync_copy(zsrc_hbm, acc)
    pltpu.sync_copy(tcnt_hbm.at[wid], cntv)
    cnt = cntv[pl.ds(0, 16)][0]
    nch = (cnt + (AC - 1)) // AC
    perblk = 8

    def load_block(j):
        src = pl.multiple_of(wid * TREG + (j // perblk) * (perblk * AC), 8)
        pltpu.sync_copy(wl_hbm.at[pl.ds(src, perblk * AC)], wbuf)

    def prep_start(slot, j):
        k0 = (j % perblk) * AC
        for k in range(AC // 16):
            w = wbuf[pl.ds(k0 + k * 16, 16)]
            pos = j * AC + k * 16 + iota16
            ok = pos < cnt
            cbuf[pl.ds(slot * AC + k * 16, 16)] = jnp.where(
                ok, w & 16383, k * 16 + iota16)
            lbuf[pl.ds(slot * AC + k * 16, 16)] = jnp.where(
                ok, w >> 14, NOWN + (iota16 & 7))
        pltpu.async_copy(hhat_hbm.at[cbuf.at[pl.ds(slot * AC, AC)]],
                         gbuf.at[slot], sem)

    @pl.when(nch > 0)
    def _():
        load_block(0)
        prep_start(0, 0)

    def body(j, carry):
        slot = j % 2
        pltpu.make_async_copy(hhat_hbm.at[cbuf.at[pl.ds(slot * AC, AC)]],
                              gbuf.at[slot], sem).wait()

        @pl.when(((j + 1) % perblk == 0) & (j + 1 < nch))
        def _():
            load_block(j + 1)

        @pl.when(j + 1 < nch)
        def _():
            prep_start((j + 1) % 2, j + 1)

        def grp(k, carry2):
            lv = lbuf[pl.ds(slot * AC + k * 16, 16)]
            for p in range(16):
                row = lv[p]
                e = k * 16 + p
                for v in range(D // 16):
                    plsc.addupdate(acc.at[row, pl.ds(v * 16, 16)],
                                   gbuf[slot, e, pl.ds(v * 16, 16)])
            return carry2

        lax.fori_loop(0, AC // 16, grp, 0)
        return carry

    lax.fori_loop(0, nch, body, 0)
    pltpu.sync_copy(acc.at[pl.ds(0, NOWN)],
                    out_hbm.at[pl.ds(pl.multiple_of(wid * NOWN, 8), NOWN)])


@functools.partial(
    pl.kernel,
    out_type=jax.ShapeDtypeStruct((NTILE, 32), f32),
    mesh=_MESH,
    compiler_params=_SC_PARAMS,
    scratch_types=[
        pltpu.VMEM((WIN,), i32),
        pltpu.VMEM((WIN,), i32),
        pltpu.VMEM((PT + 16,), i32),
        pltpu.VMEM((2 * LC,), i32),
        pltpu.VMEM((2 * LC,), i32),
        pltpu.VMEM((2, LC, D), f32),
        pltpu.VMEM((2, LC, D), f32),
        pltpu.VMEM((32,), f32),
        pltpu.SemaphoreType.DMA,
        pltpu.SemaphoreType.DMA,
    ],
)
def _loss_k(rep_hbm, pa_hbm, pb_hbm, out_hbm,
            paw, pbw, pwords, abuf, bbuf, bufa, bufb, pout, sema, semb):
    cc = lax.axis_index("c")
    ss = lax.axis_index("s")
    wid = ss * 2 + cc
    iota16 = jnp.arange(16, dtype=i32)
    ones16 = jnp.ones((16,), f32)
    zeros16 = jnp.zeros((16,), f32)

    def win(w, ptr):
        woff = pl.multiple_of(wid * PT + w * WIN, 8)
        pltpu.sync_copy(pa_hbm.at[pl.ds(woff, WIN)], paw)
        pltpu.sync_copy(pb_hbm.at[pl.ds(woff, WIN)], pbw)

        def grp(g, ptr2):
            av = paw[pl.ds(g * 16, 16)]
            bv = pbw[pl.ds(g * 16, 16)]
            gpos = wid * PT + w * WIN + g * 16 + iota16
            ok = av < bv
            word = av * 32768 + bv * 2 + jnp.where(gpos < E, 1, 0).astype(i32)
            plsc.store_compressed(pwords.at[pl.ds(ptr2, 16)], word, mask=ok)
            return ptr2 + _pop(ok)

        return lax.fori_loop(0, WIN // 16, grp, ptr)

    cnt = lax.fori_loop(0, PT // WIN, win, jnp.zeros((), i32))
    nch = (cnt + (LC - 1)) // LC

    def prep_start(slot, j):
        for k in range(LC // 16):
            w = pwords[pl.ds(j * LC + k * 16, 16)]
            pos = j * LC + k * 16 + iota16
            ok = pos < cnt
            abuf[pl.ds(slot * LC + k * 16, 16)] = jnp.where(
                ok, w >> 15, k * 16 + iota16)
            bbuf[pl.ds(slot * LC + k * 16, 16)] = jnp.where(
                ok, (w >> 1) & 16383, k * 16 + iota16)
        pltpu.async_copy(rep_hbm.at[abuf.at[pl.ds(slot * LC, LC)]],
                         bufa.at[slot], sema)
        pltpu.async_copy(rep_hbm.at[bbuf.at[pl.ds(slot * LC, LC)]],
                         bufb.at[slot], semb)

    @pl.when(nch > 0)
    def _():
        prep_start(0, 0)

    def body(j, sqc):
        slot = j % 2
        pltpu.make_async_copy(rep_hbm.at[abuf.at[pl.ds(slot * LC, LC)]],
                              bufa.at[slot], sema).wait()
        pltpu.make_async_copy(rep_hbm.at[bbuf.at[pl.ds(slot * LC, LC)]],
                              bufb.at[slot], semb).wait()

        @pl.when(j + 1 < nch)
        def _():
            prep_start((j + 1) % 2, j + 1)

        def group(g, sqc2):
            dotv = zeros16
            for p in range(16):
                row = g * 16 + p
                acc = (bufa[slot, row, pl.ds(0, 16)]
                       * bufb[slot, row, pl.ds(0, 16)])
                for v in range(1, D // 16):
                    acc = acc + (bufa[slot, row, pl.ds(v * 16, 16)]
                                 * bufb[slot, row, pl.ds(v * 16, 16)])
                dot = jnp.sum(acc)
                onehot = jnp.where(iota16 == p, f32(1.0), f32(0.0))
                dotv = dotv + dot * onehot
            w = pwords[pl.ds(j * LC + g * 16, 16)]
            pos = j * LC + g * 16 + iota16
            okv = jnp.where(pos < cnt, ones16, zeros16)
            tv = (w & 1).astype(f32)
            dv = dotv - tv
            return sqc2 + okv * dv * dv

        return lax.fori_loop(0, LC // 16, group, sqc)

    sqv = lax.fori_loop(0, nch, body, zeros16)
    pout[pl.ds(0, 16)] = sqv
    pout[pl.ds(16, 16)] = cnt.astype(f32) * jnp.where(iota16 == 0,
                                                      f32(1.0), f32(0.0))
    pltpu.sync_copy(pout, out_hbm.at[wid])


def _enc_body(deg_ref, x_ref, w_ref, b_ref, hh_ref, dinv_ref):
    deg = deg_ref[...].astype(f32)
    dinv = jnp.where(deg > 0, lax.rsqrt(deg), f32(0.0))
    h = jnp.dot(x_ref[...], w_ref[...], preferred_element_type=f32) + b_ref[...]
    hh_ref[...] = h * dinv[:, None]
    dinv_ref[...] = dinv[:, None]


def _mid_body(s1_ref, dinv_ref, w_ref, b_ref, hh_ref):
    dinv = dinv_ref[...]
    h = jnp.maximum(s1_ref[...] * dinv, f32(0.0))
    h2 = jnp.dot(h, w_ref[...], preferred_element_type=f32) + b_ref[...]
    hh_ref[...] = h2 * dinv


def _norm_body(s2_ref, dinv_ref, rep_ref):
    o = s2_ref[...] * dinv_ref[...]
    nrm = jnp.sqrt(jnp.sum(o * o, axis=1, keepdims=True))
    rep_ref[...] = o / jnp.maximum(nrm, f32(1e-12))


def _fin_body(p_ref, o_ref):
    p = p_ref[...]
    lane = lax.broadcasted_iota(i32, p.shape, 1)
    sq = jnp.sum(jnp.where(lane < 16, p, f32(0.0)))
    m = jnp.sum(jnp.where(lane >= 16, p, f32(0.0)))
    o_ref[...] = jnp.reshape(sq * f32(N) / m, (1, 1))


_enc = pl.pallas_call(
    _enc_body,
    grid=(NBLK,),
    in_specs=[
        pl.BlockSpec((RB,), lambda i: (i,)),
        pl.BlockSpec((RB, D), lambda i: (i, 0)),
        pl.BlockSpec((D, D), lambda i: (0, 0)),
        pl.BlockSpec((1, D), lambda i: (0, 0)),
    ],
    out_specs=[
        pl.BlockSpec((RB, D), lambda i: (i, 0)),
        pl.BlockSpec((RB, 1), lambda i: (i, 0)),
    ],
    out_shape=[
        jax.ShapeDtypeStruct((NP, D), f32),
        jax.ShapeDtypeStruct((NP, 1), f32),
    ],
)

_mid = pl.pallas_call(
    _mid_body,
    grid=(NBLK,),
    in_specs=[
        pl.BlockSpec((RB, D), lambda i: (i, 0)),
        pl.BlockSpec((RB, 1), lambda i: (i, 0)),
        pl.BlockSpec((D, D), lambda i: (0, 0)),
        pl.BlockSpec((1, D), lambda i: (0, 0)),
    ],
    out_specs=pl.BlockSpec((RB, D), lambda i: (i, 0)),
    out_shape=jax.ShapeDtypeStruct((NP, D), f32),
)

_normk = pl.pallas_call(
    _norm_body,
    grid=(NBLK,),
    in_specs=[
        pl.BlockSpec((RB, D), lambda i: (i, 0)),
        pl.BlockSpec((RB, 1), lambda i: (i, 0)),
    ],
    out_specs=pl.BlockSpec((RB, D), lambda i: (i, 0)),
    out_shape=jax.ShapeDtypeStruct((NP, D), f32),
)

_fin = pl.pallas_call(
    _fin_body,
    in_specs=[pl.BlockSpec((NTILE, 32), lambda: (0, 0))],
    out_specs=pl.BlockSpec((1, 1), lambda: (0, 0)),
    out_shape=jax.ShapeDtypeStruct((1, 1), f32),
)


def kernel(features, edge_index, W1, b1, W2, b2):
    x = jnp.pad(features, ((0, NP - N), (0, 0)))
    loop = jnp.arange(N, dtype=i32)
    epad = EPAD - EALL
    rows = jnp.concatenate(
        [edge_index[0], loop, jnp.full((epad,), -1, i32)])
    cols = jnp.concatenate(
        [edge_index[1], loop, jnp.zeros((epad,), i32)])
    zsrc = jnp.zeros((ACC2, D), f32)

    wl, tcnt, deg = _route_k(rows, cols)
    hhat1, dinv = _enc(deg, x, W1, b1.reshape(1, D))
    s1 = _agg_k(hhat1, wl, tcnt, zsrc)
    hhat2 = _mid(s1, dinv, W2, b2.reshape(1, D))
    s2 = _agg_k(hhat2, wl, tcnt, zsrc)
    rep_full = _normk(s2, dinv)

    randn = jax.random.randint(jax.random.key(42), (2, NEG), 0, N, dtype=i32)
    ppad = PPAD - NPAIR
    spread = (jnp.arange(ppad, dtype=i32) * 53) % N
    pa = jnp.concatenate([edge_index[0], randn[0], spread])
    pb = jnp.concatenate([edge_index[1], randn[1], spread])

    partials = _loss_k(rep_full, pa, pb)
    loss = _fin(partials)[0, 0]
    return rep_full[:N], loss
